# Initial kernel scaffold; baseline (speedup 1.0000x reference)
import functools

import jax
import jax.numpy as jnp
from jax import lax
from jax.experimental import pallas as pl
from jax.experimental.pallas import tpu as pltpu
from jax.experimental.pallas import tpu_sc as plsc

_info = plsc.get_sparse_core_info()
_NC, _NS = _info.num_cores, _info.num_subcores
_NW = _NC * _NS

_K = 10
_CHUNK = _K * 128


def _sc_gather(table, idx2d):
  n_rows128, _ = idx2d.shape
  n_total = n_rows128 * 128
  d = table.shape[1]
  rows_per_w = n_rows128 // _NW
  n_chunks = rows_per_w // _K

  mesh = plsc.VectorSubcoreMesh(core_axis_name="c", subcore_axis_name="s")

  @functools.partial(
      pl.kernel,
      mesh=mesh,
      out_type=jax.ShapeDtypeStruct((n_total, d), jnp.float32),
      scratch_types=[
          pltpu.VMEM((_K, 128), jnp.int32),
          pltpu.VMEM((_CHUNK, d), jnp.float32),
          pltpu.SemaphoreType.DMA,
      ],
  )
  def k(table_hbm, idx_hbm, out_hbm, idx_v, rows_v, sem):
    wid = lax.axis_index("s") * _NC + lax.axis_index("c")
    row0 = wid * rows_per_w

    def body(g, carry):
      r0 = row0 + g * _K
      pltpu.sync_copy(idx_hbm.at[pl.ds(r0, _K), :], idx_v)
      copies = []
      for j in range(_K):
        copies.append(
            pltpu.async_copy(
                table_hbm.at[idx_v.at[j]],
                rows_v.at[pl.ds(j * 128, 128), :],
                sem,
            ))
      for c in copies:
        c.wait()
      pltpu.sync_copy(rows_v, out_hbm.at[pl.ds(r0 * 128, _CHUNK), :])
      return carry

    lax.fori_loop(0, n_chunks, body, 0)

  return k(table, idx2d)


def _tc_transform(gv, w4, b4, a):
  m = gv.shape[0]
  blk = 2048

  def body(g_ref, w_ref, b_ref, a_ref, o_ref):
    y = jnp.dot(g_ref[...], w_ref[...], preferred_element_type=jnp.float32)
    y = y + b_ref[...]
    alpha = a_ref[0]
    o_ref[...] = jnp.where(y >= 0, y, alpha * y)

  return pl.pallas_call(
      body,
      grid=(m // blk,),
      in_specs=[
          pl.BlockSpec((blk, 128), lambda i: (i, 0)),
          pl.BlockSpec((128, 128), lambda i: (0, 0)),
          pl.BlockSpec((1, 128), lambda i: (0, 0)),
          pl.BlockSpec(memory_space=pltpu.SMEM),
      ],
      out_specs=pl.BlockSpec((blk, 128), lambda i: (i, 0)),
      out_shape=jax.ShapeDtypeStruct((m, 128), jnp.float32),
  )(gv, w4, b4, a)


def kernel(x, table, W, b, a):
  bsz, hist = x.shape
  d = table.shape[1]
  n_total = bsz * hist

  idx2d = x.reshape(n_total // 128, 128).astype(jnp.int32)
  gathered = _sc_gather(table, idx2d)

  rep = 128 // d
  w4 = jnp.kron(jnp.eye(rep, dtype=W.dtype), W.T)
  b4 = jnp.tile(b, rep).reshape(1, 128)

  gv = gathered.reshape(n_total * d // 128, 128)
  out = _tc_transform(gv, w4, b4, a)
  return out.reshape(bsz, hist, d)


# speedup vs baseline: 18.0639x; 18.0639x over previous
import functools

import jax
import jax.numpy as jnp
from jax import lax
from jax.experimental import pallas as pl
from jax.experimental.pallas import tpu as pltpu
from jax.experimental.pallas import tpu_sc as plsc

_info = plsc.get_sparse_core_info()
_NC, _NS = _info.num_cores, _info.num_subcores
_NW = _NC * _NS

_K = 8
_CHUNK = _K * 128


def _sc_gather(table, idx2d):
  n_rows128, _ = idx2d.shape
  n_total = n_rows128 * 128
  d = table.shape[1]
  rows_per_w = n_rows128 // _NW
  n_chunks = rows_per_w // _K

  mesh = plsc.VectorSubcoreMesh(core_axis_name="c", subcore_axis_name="s")

  @functools.partial(
      pl.kernel,
      mesh=mesh,
      out_type=jax.ShapeDtypeStruct((n_total, d), jnp.float32),
      scratch_types=[
          pltpu.VMEM((_K, 128), jnp.int32),
          pltpu.VMEM((_CHUNK, d), jnp.float32),
          pltpu.SemaphoreType.DMA,
      ],
      compiler_params=pltpu.CompilerParams(use_tc_tiling_on_sc=False),
  )
  def k(table_hbm, idx_hbm, out_hbm, idx_v, rows_v, sem):
    wid = lax.axis_index("s") * _NC + lax.axis_index("c")
    row0 = wid * rows_per_w

    def body(g, carry):
      r0 = row0 + g * _K
      pltpu.sync_copy(idx_hbm.at[pl.ds(r0, _K), :], idx_v)
      copies = []
      for j in range(_K):
        copies.append(
            pltpu.async_copy(
                table_hbm.at[idx_v.at[j]],
                rows_v.at[pl.ds(j * 128, 128), :],
                sem,
            ))
      for c in copies:
        c.wait()
      pltpu.sync_copy(rows_v, out_hbm.at[pl.ds(r0 * 128, _CHUNK), :])
      return carry

    lax.fori_loop(0, n_chunks, body, 0)

  return k(table, idx2d)


def _tc_transform(gv, w4, b4, a):
  m = gv.shape[0]
  blk = 2048

  def body(g_ref, w_ref, b_ref, a_ref, o_ref):
    y = jnp.dot(g_ref[...], w_ref[...], preferred_element_type=jnp.float32)
    y = y + b_ref[...]
    alpha = a_ref[0]
    o_ref[...] = jnp.where(y >= 0, y, alpha * y)

  return pl.pallas_call(
      body,
      grid=(m // blk,),
      in_specs=[
          pl.BlockSpec((blk, 128), lambda i: (i, 0)),
          pl.BlockSpec((128, 128), lambda i: (0, 0)),
          pl.BlockSpec((1, 128), lambda i: (0, 0)),
          pl.BlockSpec(memory_space=pltpu.SMEM),
      ],
      out_specs=pl.BlockSpec((blk, 128), lambda i: (i, 0)),
      out_shape=jax.ShapeDtypeStruct((m, 128), jnp.float32),
  )(gv, w4, b4, a)


def kernel(x, table, W, b, a):
  bsz, hist = x.shape
  d = table.shape[1]
  n_total = bsz * hist

  idx2d = x.reshape(n_total // 128, 128).astype(jnp.int32)
  gathered = _sc_gather(table, idx2d)

  rep = 128 // d
  w4 = jnp.kron(jnp.eye(rep, dtype=W.dtype), W.T)
  b4 = jnp.tile(b, rep).reshape(1, 128)

  gv = gathered.reshape(n_total * d // 128, 128)
  out = _tc_transform(gv, w4, b4, a)
  return out.reshape(bsz, hist, d)


# speedup vs baseline: 23.1556x; 1.2819x over previous
import functools

import jax
import jax.numpy as jnp
from jax import lax
from jax.experimental import pallas as pl
from jax.experimental.pallas import tpu as pltpu
from jax.experimental.pallas import tpu_sc as plsc

_info = plsc.get_sparse_core_info()
_NC, _NS = _info.num_cores, _info.num_subcores
_NW = _NC * _NS

_K = 8
_CHUNK = _K * 128


def _sc_gather(table, idx2d):
  n_rows128, _ = idx2d.shape
  n_total = n_rows128 * 128
  d = table.shape[1]
  rows_per_w = n_rows128 // _NW
  n_chunks = rows_per_w // _K

  mesh = plsc.VectorSubcoreMesh(core_axis_name="c", subcore_axis_name="s")

  @functools.partial(
      pl.kernel,
      mesh=mesh,
      out_type=jax.ShapeDtypeStruct((n_total, d), jnp.float32),
      scratch_types=[
          pltpu.VMEM((_K, 128), jnp.int32),
          pltpu.VMEM((_CHUNK, d), jnp.float32),
          pltpu.SemaphoreType.DMA,
      ],
      compiler_params=pltpu.CompilerParams(use_tc_tiling_on_sc=False),
  )
  def k(table_hbm, idx_hbm, out_hbm, idx_v, rows_v, sem):
    wid = lax.axis_index("s") * _NC + lax.axis_index("c")
    row0 = wid * rows_per_w

    def body(g, carry):
      r0 = row0 + g * _K
      pltpu.sync_copy(idx_hbm.at[pl.ds(r0, _K), :], idx_v)
      copies = []
      for j in range(_K):
        copies.append(
            pltpu.async_copy(
                table_hbm.at[idx_v.at[j]],
                rows_v.at[pl.ds(j * 128, 128), :],
                sem,
            ))
      for c in copies:
        c.wait()
      pltpu.sync_copy(rows_v, out_hbm.at[pl.ds(r0 * 128, _CHUNK), :])
      return carry

    lax.fori_loop(0, n_chunks, body, 0)

  return k(table, idx2d)


def _tc_transform_t(gv3, w4, b128, a, hist, bsz, d):
  rep = 128 // d
  bblk = 2048
  vrows = bblk * d // 128

  def body(g_ref, w_ref, b_ref, a_ref, o_ref):
    e_packed = g_ref[0]
    et = jax.lax.transpose(e_packed, (1, 0))
    y = jax.lax.dot_general(
        w_ref[...], et, (((1,), (0,)), ((), ())),
        preferred_element_type=jnp.float32)
    y = y + b_ref[...]
    alpha = a_ref[0]
    y = jnp.where(y >= 0, y, alpha * y)
    o_ref[0] = jnp.concatenate(
        [y[q * d:(q + 1) * d, :] for q in range(rep)], axis=1)

  return pl.pallas_call(
      body,
      grid=(hist, bsz // bblk),
      in_specs=[
          pl.BlockSpec((1, vrows, 128), lambda l, i: (l, i, 0)),
          pl.BlockSpec((128, 128), lambda l, i: (0, 0)),
          pl.BlockSpec((128, 1), lambda l, i: (0, 0)),
          pl.BlockSpec(memory_space=pltpu.SMEM),
      ],
      out_specs=pl.BlockSpec((1, d, bblk), lambda l, i: (l, 0, i)),
      out_shape=jax.ShapeDtypeStruct((hist, d, bsz), jnp.float32),
  )(gv3, w4, b128, a)


def kernel(x, table, W, b, a):
  bsz, hist = x.shape
  d = table.shape[1]
  rep = 128 // d
  n_total = bsz * hist

  sub = 2048 // rep
  idxp = (x.T.astype(jnp.int32)
          .reshape(hist, bsz // 2048, rep, sub)
          .transpose(0, 1, 3, 2))
  idx2d = idxp.reshape(n_total // 128, 128)
  gathered = _sc_gather(table, idx2d)

  w4 = jnp.kron(jnp.eye(rep, dtype=W.dtype), W)
  b128 = jnp.tile(b, rep).reshape(128, 1)

  gv3 = gathered.reshape(hist, bsz * d // 128, 128)
  out_t = _tc_transform_t(gv3, w4, b128, a, hist, bsz, d)
  return out_t.transpose(2, 0, 1)
